# baseline (speedup 1.0000x reference)
import jax
import jax.numpy as jnp
from jax.experimental import pallas as pl


_E_BLOCK = 512


def _cos_body(a_ref, b_ref, o_ref):
    a = a_ref[...]
    b = b_ref[...]
    dot = jnp.sum(a * b, axis=1)
    ns = jnp.sqrt(jnp.sum(a * a, axis=1))
    nd = jnp.sqrt(jnp.sum(b * b, axis=1))
    o_ref[...] = dot / jnp.maximum(ns * nd, 1e-8)


def kernel(h, edge_index):
    ei = edge_index.astype(jnp.int32)
    hs = jnp.take(h, ei[0], axis=0)
    hd = jnp.take(h, ei[1], axis=0)
    e = hs.shape[0]
    nblk = e // _E_BLOCK
    out = pl.pallas_call(
        _cos_body,
        grid=(nblk,),
        in_specs=[
            pl.BlockSpec((_E_BLOCK, 128), lambda i: (i, 0)),
            pl.BlockSpec((_E_BLOCK, 128), lambda i: (i, 0)),
        ],
        out_specs=pl.BlockSpec((_E_BLOCK,), lambda i: (i,)),
        out_shape=jax.ShapeDtypeStruct((e,), jnp.float32),
    )(hs, hd)
    return out


# speedup vs baseline: 4.4246x; 4.4246x over previous
import functools

import jax
import jax.numpy as jnp
from jax import lax
from jax.experimental import pallas as pl
from jax.experimental.pallas import tpu as pltpu
from jax.experimental.pallas import tpu_sc as plsc

N_NODES_ = 10000
N_EDGES_ = 320000
D_ = 128
NW_ = 32
PER_TILE_ = N_EDGES_ // NW_
C_ = 80
NCHUNK_ = PER_TILE_ // C_
EPS_ = 1e-8


def _norm_body(h_ref, n_ref):
    x = h_ref[...]
    n_ref[...] = jnp.sqrt(jnp.sum(x * x, axis=1))


def _node_norms(h):
    return pl.pallas_call(
        _norm_body,
        out_shape=jax.ShapeDtypeStruct((N_NODES_,), jnp.float32),
    )(h)


def _sc_body(h_hbm, src_hbm, dst_hbm, n_hbm, out_hbm,
             idx_s, idx_d, ntab, buf_a, buf_b, outv, sem_a, sem_b):
    cid = lax.axis_index("c")
    sid = lax.axis_index("s")
    wid = sid * 2 + cid
    base = wid * PER_TILE_

    pltpu.sync_copy(src_hbm.at[pl.ds(base, PER_TILE_)], idx_s)
    pltpu.sync_copy(dst_hbm.at[pl.ds(base, PER_TILE_)], idx_d)
    pltpu.sync_copy(n_hbm, ntab)

    lane = lax.broadcasted_iota(jnp.int32, (16,), 0)

    def chunk_body(ci, _):
        off = ci * C_
        cp_a = pltpu.async_copy(h_hbm.at[idx_s.at[pl.ds(off, C_)]], buf_a, sem_a)
        cp_b = pltpu.async_copy(h_hbm.at[idx_d.at[pl.ds(off, C_)]], buf_b, sem_b)
        cp_a.wait()
        cp_b.wait()

        def group_body(g, _):
            acc16 = jnp.zeros((16,), jnp.float32)
            for e16 in range(16):
                ec = g * 16 + e16
                acc = buf_a[ec, pl.ds(0, 16)] * buf_b[ec, pl.ds(0, 16)]
                for j in range(1, 8):
                    acc = acc + buf_a[ec, pl.ds(16 * j, 16)] * buf_b[ec, pl.ds(16 * j, 16)]
                for sh in (8, 4, 2, 1):
                    acc = acc + acc.at[lane ^ sh].get(mode="promise_in_bounds")
                acc16 = jnp.where(lane == e16, acc, acc16)
            outv[pl.ds(off + g * 16, 16)] = acc16
            return 0

        lax.fori_loop(0, C_ // 16, group_body, 0)
        return 0

    lax.fori_loop(0, NCHUNK_, chunk_body, 0)

    def den_body(i, _):
        sl = pl.ds(i * 16, 16)
        d16 = outv[sl]
        ns = plsc.load_gather(ntab, [idx_s[sl]])
        nd = plsc.load_gather(ntab, [idx_d[sl]])
        outv[sl] = d16 / jnp.maximum(ns * nd, EPS_)
        return 0

    lax.fori_loop(0, PER_TILE_ // 16, den_body, 0)

    pltpu.sync_copy(outv, out_hbm.at[pl.ds(base, PER_TILE_)])


def _edge_cos_sc(h, src, dst, n):
    mesh = plsc.VectorSubcoreMesh(core_axis_name="c", subcore_axis_name="s")
    f = functools.partial(
        pl.kernel,
        mesh=mesh,
        out_type=jax.ShapeDtypeStruct((N_EDGES_,), jnp.float32),
        scratch_types=[
            pltpu.VMEM((PER_TILE_,), jnp.int32),
            pltpu.VMEM((PER_TILE_,), jnp.int32),
            pltpu.VMEM((N_NODES_,), jnp.float32),
            pltpu.VMEM((C_, D_), jnp.float32),
            pltpu.VMEM((C_, D_), jnp.float32),
            pltpu.VMEM((PER_TILE_,), jnp.float32),
            pltpu.SemaphoreType.DMA,
            pltpu.SemaphoreType.DMA,
        ],
        compiler_params=pltpu.CompilerParams(needs_layout_passes=False),
    )(_sc_body)
    return f(h, src, dst, n)


def kernel(h, edge_index):
    ei = edge_index.astype(jnp.int32)
    n = _node_norms(h)
    return _edge_cos_sc(h, ei[0], ei[1], n)


# speedup vs baseline: 4.5533x; 1.0291x over previous
import functools

import jax
import jax.numpy as jnp
from jax import lax
from jax.experimental import pallas as pl
from jax.experimental.pallas import tpu as pltpu
from jax.experimental.pallas import tpu_sc as plsc

N_NODES_ = 10000
N_EDGES_ = 320000
D_ = 128
NW_ = 32
PER_TILE_ = N_EDGES_ // NW_
C_ = 80
NCHUNK_ = PER_TILE_ // C_
EPS_ = 1e-8


def _norm_body(h_ref, n_ref):
    x = h_ref[...]
    n_ref[...] = jnp.sqrt(jnp.sum(x * x, axis=1))


def _node_norms(h):
    return pl.pallas_call(
        _norm_body,
        out_shape=jax.ShapeDtypeStruct((N_NODES_,), jnp.float32),
    )(h)


def _sc_body(h_hbm, src_hbm, dst_hbm, n_hbm, out_hbm,
             idx_s, idx_d, ntab, ba0, bb0, ba1, bb1, outv,
             sa0, sb0, sa1, sb1):
    cid = lax.axis_index("c")
    sid = lax.axis_index("s")
    wid = sid * 2 + cid
    base = wid * PER_TILE_

    pltpu.sync_copy(src_hbm.at[pl.ds(base, PER_TILE_)], idx_s)
    pltpu.sync_copy(dst_hbm.at[pl.ds(base, PER_TILE_)], idx_d)
    pltpu.sync_copy(n_hbm, ntab)

    lane = lax.broadcasted_iota(jnp.int32, (16,), 0)
    bufs = ((ba0, bb0, sa0, sb0), (ba1, bb1, sa1, sb1))

    def fire(k, slot):
        buf_a, buf_b, sem_a, sem_b = bufs[slot]
        off = k * C_
        pltpu.async_copy(h_hbm.at[idx_s.at[pl.ds(off, C_)]], buf_a, sem_a)
        pltpu.async_copy(h_hbm.at[idx_d.at[pl.ds(off, C_)]], buf_b, sem_b)

    def compute(k, slot):
        buf_a, buf_b, sem_a, sem_b = bufs[slot]
        pltpu.make_async_copy(h_hbm.at[pl.ds(0, C_)], buf_a, sem_a).wait()
        pltpu.make_async_copy(h_hbm.at[pl.ds(0, C_)], buf_b, sem_b).wait()
        off = k * C_
        for g in range(C_ // 16):
            acc16 = jnp.zeros((16,), jnp.float32)
            for e16 in range(16):
                ec = g * 16 + e16
                acc = buf_a[ec, pl.ds(0, 16)] * buf_b[ec, pl.ds(0, 16)]
                for j in range(1, 8):
                    acc = acc + (buf_a[ec, pl.ds(16 * j, 16)]
                                 * buf_b[ec, pl.ds(16 * j, 16)])
                for sh in (8, 4, 2, 1):
                    acc = acc + acc.at[lane ^ sh].get(mode="promise_in_bounds")
                acc16 = jnp.where(lane == e16, acc, acc16)
            outv[pl.ds(off + g * 16, 16)] = acc16

    fire(0, 0)

    def pair_body(i, _):
        k0 = i * 2
        fire(k0 + 1, 1)
        compute(k0, 0)
        fire(k0 + 2, 0)
        compute(k0 + 1, 1)
        return 0

    lax.fori_loop(0, (NCHUNK_ - 1) // 2, pair_body, 0)
    compute(NCHUNK_ - 1, 0)

    def den_body(i, _):
        sl = pl.ds(i * 16, 16)
        d16 = outv[sl]
        ns = plsc.load_gather(ntab, [idx_s[sl]])
        nd = plsc.load_gather(ntab, [idx_d[sl]])
        outv[sl] = d16 / jnp.maximum(ns * nd, EPS_)
        return 0

    lax.fori_loop(0, PER_TILE_ // 16, den_body, 0)

    pltpu.sync_copy(outv, out_hbm.at[pl.ds(base, PER_TILE_)])


def _edge_cos_sc(h, src, dst, n):
    mesh = plsc.VectorSubcoreMesh(core_axis_name="c", subcore_axis_name="s")
    f = functools.partial(
        pl.kernel,
        mesh=mesh,
        out_type=jax.ShapeDtypeStruct((N_EDGES_,), jnp.float32),
        scratch_types=[
            pltpu.VMEM((PER_TILE_,), jnp.int32),
            pltpu.VMEM((PER_TILE_,), jnp.int32),
            pltpu.VMEM((N_NODES_,), jnp.float32),
            pltpu.VMEM((C_, D_), jnp.float32),
            pltpu.VMEM((C_, D_), jnp.float32),
            pltpu.VMEM((C_, D_), jnp.float32),
            pltpu.VMEM((C_, D_), jnp.float32),
            pltpu.VMEM((PER_TILE_,), jnp.float32),
            pltpu.SemaphoreType.DMA,
            pltpu.SemaphoreType.DMA,
            pltpu.SemaphoreType.DMA,
            pltpu.SemaphoreType.DMA,
        ],
        compiler_params=pltpu.CompilerParams(needs_layout_passes=False),
    )(_sc_body)
    return f(h, src, dst, n)


def kernel(h, edge_index):
    ei = edge_index.astype(jnp.int32)
    n = _node_norms(h)
    return _edge_cos_sc(h, ei[0], ei[1], n)


# speedup vs baseline: 5.3666x; 1.1786x over previous
import functools

import jax
import jax.numpy as jnp
from jax import lax
from jax.experimental import pallas as pl
from jax.experimental.pallas import tpu as pltpu
from jax.experimental.pallas import tpu_sc as plsc

N_NODES_ = 10000
N_EDGES_ = 320000
D_ = 128
NW_ = 32
PER_TILE_ = N_EDGES_ // NW_
C_ = 48
NCHUNK_ = 210
PAD_TILE_ = NCHUNK_ * C_
EPS_ = 1e-8


def _norm_body(h_ref, nsq_ref, n_ref):
    x = h_ref[...]
    s = jnp.sum(x * x, axis=1)
    nsq_ref[...] = s
    n_ref[...] = jnp.sqrt(s)


def _node_norms(h):
    return pl.pallas_call(
        _norm_body,
        out_shape=[
            jax.ShapeDtypeStruct((N_NODES_,), jnp.float32),
            jax.ShapeDtypeStruct((N_NODES_,), jnp.float32),
        ],
    )(h)


def _sc_body(h_hbm, src_hbm, dst_hbm, nsq_hbm, n_hbm, out_hbm,
             idx_s, idx_d, nsqt, ntab, buf0, buf1, buf2, outv,
             sb0, sb1, sb2, sa0, sa1, sa2):
    cid = lax.axis_index("c")
    sid = lax.axis_index("s")
    wid = sid * 2 + cid
    base = wid * PER_TILE_

    pltpu.sync_copy(src_hbm.at[pl.ds(base, PER_TILE_)], idx_s.at[pl.ds(0, PER_TILE_)])
    pltpu.sync_copy(dst_hbm.at[pl.ds(base, PER_TILE_)], idx_d.at[pl.ds(0, PER_TILE_)])
    pltpu.sync_copy(nsq_hbm, nsqt)
    pltpu.sync_copy(n_hbm, ntab)

    zeros_i = jnp.zeros((16,), jnp.int32)
    for t in range((PAD_TILE_ - PER_TILE_) // 16):
        idx_s[pl.ds(PER_TILE_ + 16 * t, 16)] = zeros_i
        idx_d[pl.ds(PER_TILE_ + 16 * t, 16)] = zeros_i

    lane = lax.broadcasted_iota(jnp.int32, (16,), 0)
    bufs = (buf0, buf1, buf2)
    semb = (sb0, sb1, sb2)
    sema = (sa0, sa1, sa2)

    def fire_base(k, slot):
        pltpu.async_copy(h_hbm.at[idx_s.at[pl.ds(k * C_, C_)]],
                         bufs[slot], semb[slot])

    def fire_add(k, slot):
        pltpu.async_copy(h_hbm.at[idx_d.at[pl.ds(k * C_, C_)]],
                         bufs[slot], sema[slot], add=True)

    def wait_base(slot):
        pltpu.make_async_copy(h_hbm.at[pl.ds(0, C_)], bufs[slot],
                              semb[slot]).wait()

    def wait_add(slot):
        pltpu.make_async_copy(h_hbm.at[pl.ds(0, C_)], bufs[slot],
                              sema[slot]).wait()

    def compute(k, slot):
        buf = bufs[slot]
        off = k * C_
        for g in range(C_ // 16):
            ssq16 = jnp.zeros((16,), jnp.float32)
            for e16 in range(16):
                ec = g * 16 + e16
                v = buf[ec, pl.ds(0, 16)]
                acc = v * v
                for j in range(1, 8):
                    v = buf[ec, pl.ds(16 * j, 16)]
                    acc = acc + v * v
                for sh in (8, 4, 2, 1):
                    acc = acc + acc.at[lane ^ sh].get(mode="promise_in_bounds")
                ssq16 = jnp.where(lane == e16, acc, ssq16)
            eoff = off + g * 16
            is16 = idx_s[pl.ds(eoff, 16)]
            id16 = idx_d[pl.ds(eoff, 16)]
            nsqs = plsc.load_gather(nsqt, [is16])
            nsqd = plsc.load_gather(nsqt, [id16])
            ns = plsc.load_gather(ntab, [is16])
            nd = plsc.load_gather(ntab, [id16])
            dot16 = (ssq16 - nsqs - nsqd) * 0.5
            outv[pl.ds(eoff, 16)] = dot16 / jnp.maximum(ns * nd, EPS_)

    fire_base(0, 0)
    fire_base(1, 1)
    fire_base(2, 2)
    wait_base(0)
    fire_add(0, 0)

    def triple_body(i, _):
        k0 = i * 3
        for b in range(3):
            k = k0 + b
            nslot = (b + 1) % 3

            @pl.when(k + 1 < NCHUNK_)
            def _():
                wait_base(nslot)
                fire_add(k + 1, nslot)

            wait_add(b)
            compute(k, b)

            @pl.when(k + 3 < NCHUNK_)
            def _():
                fire_base(k + 3, b)
        return 0

    lax.fori_loop(0, NCHUNK_ // 3, triple_body, 0)

    pltpu.sync_copy(outv.at[pl.ds(0, PER_TILE_)],
                    out_hbm.at[pl.ds(base, PER_TILE_)])


def _edge_cos_sc(h, src, dst, nsq, n):
    mesh = plsc.VectorSubcoreMesh(core_axis_name="c", subcore_axis_name="s")
    f = functools.partial(
        pl.kernel,
        mesh=mesh,
        out_type=jax.ShapeDtypeStruct((N_EDGES_,), jnp.float32),
        scratch_types=[
            pltpu.VMEM((PAD_TILE_,), jnp.int32),
            pltpu.VMEM((PAD_TILE_,), jnp.int32),
            pltpu.VMEM((N_NODES_,), jnp.float32),
            pltpu.VMEM((N_NODES_,), jnp.float32),
            pltpu.VMEM((C_, D_), jnp.float32),
            pltpu.VMEM((C_, D_), jnp.float32),
            pltpu.VMEM((C_, D_), jnp.float32),
            pltpu.VMEM((PAD_TILE_,), jnp.float32),
            pltpu.SemaphoreType.DMA,
            pltpu.SemaphoreType.DMA,
            pltpu.SemaphoreType.DMA,
            pltpu.SemaphoreType.DMA,
            pltpu.SemaphoreType.DMA,
            pltpu.SemaphoreType.DMA,
        ],
        compiler_params=pltpu.CompilerParams(needs_layout_passes=False),
    )(_sc_body)
    return f(h, src, dst, nsq, n)


def kernel(h, edge_index):
    ei = edge_index.astype(jnp.int32)
    nsq, n = _node_norms(h)
    return _edge_cos_sc(h, ei[0], ei[1], nsq, n)


# speedup vs baseline: 5.4588x; 1.0172x over previous
import functools

import jax
import jax.numpy as jnp
from jax import lax
from jax.experimental import pallas as pl
from jax.experimental.pallas import tpu as pltpu
from jax.experimental.pallas import tpu_sc as plsc

N_NODES_ = 10000
N_EDGES_ = 320000
D_ = 128
NW_ = 32
PER_TILE_ = N_EDGES_ // NW_
C_ = 48
NCHUNK_ = 210
NSLOT_ = 5
PAD_TILE_ = NCHUNK_ * C_
EPS_ = 1e-8


def _norm_body(h_ref, nsq_ref, n_ref):
    x = h_ref[...]
    s = jnp.sum(x * x, axis=1)
    nsq_ref[...] = s
    n_ref[...] = jnp.sqrt(s)


def _node_norms(h):
    return pl.pallas_call(
        _norm_body,
        out_shape=[
            jax.ShapeDtypeStruct((N_NODES_,), jnp.float32),
            jax.ShapeDtypeStruct((N_NODES_,), jnp.float32),
        ],
    )(h)


def _sc_body(h_hbm, src_hbm, dst_hbm, nsq_hbm, n_hbm, out_hbm,
             idx_s, idx_d, nsqt, ntab, buf0, buf1, buf2, buf3, buf4, outv,
             sb0, sb1, sb2, sb3, sb4, sa0, sa1, sa2, sa3, sa4):
    cid = lax.axis_index("c")
    sid = lax.axis_index("s")
    wid = sid * 2 + cid
    base = wid * PER_TILE_

    pltpu.sync_copy(src_hbm.at[pl.ds(base, PER_TILE_)], idx_s.at[pl.ds(0, PER_TILE_)])
    pltpu.sync_copy(dst_hbm.at[pl.ds(base, PER_TILE_)], idx_d.at[pl.ds(0, PER_TILE_)])
    pltpu.sync_copy(nsq_hbm, nsqt)
    pltpu.sync_copy(n_hbm, ntab)

    zeros_i = jnp.zeros((16,), jnp.int32)
    for t in range((PAD_TILE_ - PER_TILE_) // 16):
        idx_s[pl.ds(PER_TILE_ + 16 * t, 16)] = zeros_i
        idx_d[pl.ds(PER_TILE_ + 16 * t, 16)] = zeros_i

    lane = lax.broadcasted_iota(jnp.int32, (16,), 0)
    bufs = (buf0, buf1, buf2, buf3, buf4)
    semb = (sb0, sb1, sb2, sb3, sb4)
    sema = (sa0, sa1, sa2, sa3, sa4)

    def fire_base(k, slot):
        pltpu.async_copy(h_hbm.at[idx_s.at[pl.ds(k * C_, C_)]],
                         bufs[slot], semb[slot])

    def fire_add(k, slot):
        pltpu.async_copy(h_hbm.at[idx_d.at[pl.ds(k * C_, C_)]],
                         bufs[slot], sema[slot], add=True)

    def wait_base(slot):
        pltpu.make_async_copy(h_hbm.at[pl.ds(0, C_)], bufs[slot],
                              semb[slot]).wait()

    def wait_add(slot):
        pltpu.make_async_copy(h_hbm.at[pl.ds(0, C_)], bufs[slot],
                              sema[slot]).wait()

    def compute(k, slot):
        buf = bufs[slot]
        off = k * C_
        for g in range(C_ // 16):
            ssq16 = jnp.zeros((16,), jnp.float32)
            for e16 in range(16):
                ec = g * 16 + e16
                v = buf[ec, pl.ds(0, 16)]
                acc = v * v
                for j in range(1, 8):
                    v = buf[ec, pl.ds(16 * j, 16)]
                    acc = acc + v * v
                for sh in (8, 4, 2, 1):
                    acc = acc + acc.at[lane ^ sh].get(mode="promise_in_bounds")
                ssq16 = jnp.where(lane == e16, acc, ssq16)
            eoff = off + g * 16
            is16 = idx_s[pl.ds(eoff, 16)]
            id16 = idx_d[pl.ds(eoff, 16)]
            nsqs = plsc.load_gather(nsqt, [is16])
            nsqd = plsc.load_gather(nsqt, [id16])
            ns = plsc.load_gather(ntab, [is16])
            nd = plsc.load_gather(ntab, [id16])
            dot16 = (ssq16 - nsqs - nsqd) * 0.5
            outv[pl.ds(eoff, 16)] = dot16 / jnp.maximum(ns * nd, EPS_)

    for p in range(NSLOT_):
        fire_base(p, p)
    wait_base(0)
    fire_add(0, 0)
    wait_base(1)
    fire_add(1, 1)

    def ring_body(i, _):
        k0 = i * NSLOT_
        for b in range(NSLOT_):
            k = k0 + b
            nslot = (b + 2) % NSLOT_

            @pl.when(k + 2 < NCHUNK_)
            def _():
                wait_base(nslot)
                fire_add(k + 2, nslot)

            wait_add(b)
            compute(k, b)

            @pl.when(k + NSLOT_ < NCHUNK_)
            def _():
                fire_base(k + NSLOT_, b)
        return 0

    lax.fori_loop(0, NCHUNK_ // NSLOT_, ring_body, 0)

    pltpu.sync_copy(outv.at[pl.ds(0, PER_TILE_)],
                    out_hbm.at[pl.ds(base, PER_TILE_)])


def _edge_cos_sc(h, src, dst, nsq, n):
    mesh = plsc.VectorSubcoreMesh(core_axis_name="c", subcore_axis_name="s")
    f = functools.partial(
        pl.kernel,
        mesh=mesh,
        out_type=jax.ShapeDtypeStruct((N_EDGES_,), jnp.float32),
        scratch_types=[
            pltpu.VMEM((PAD_TILE_,), jnp.int32),
            pltpu.VMEM((PAD_TILE_,), jnp.int32),
            pltpu.VMEM((N_NODES_,), jnp.float32),
            pltpu.VMEM((N_NODES_,), jnp.float32),
            pltpu.VMEM((C_, D_), jnp.float32),
            pltpu.VMEM((C_, D_), jnp.float32),
            pltpu.VMEM((C_, D_), jnp.float32),
            pltpu.VMEM((C_, D_), jnp.float32),
            pltpu.VMEM((C_, D_), jnp.float32),
            pltpu.VMEM((PAD_TILE_,), jnp.float32),
        ] + [pltpu.SemaphoreType.DMA] * 10,
        compiler_params=pltpu.CompilerParams(needs_layout_passes=False),
    )(_sc_body)
    return f(h, src, dst, nsq, n)


def kernel(h, edge_index):
    ei = edge_index.astype(jnp.int32)
    nsq, n = _node_norms(h)
    return _edge_cos_sc(h, ei[0], ei[1], nsq, n)


# speedup vs baseline: 12.7759x; 2.3404x over previous
import functools

import jax
import jax.numpy as jnp
from jax import lax
from jax.experimental import pallas as pl
from jax.experimental.pallas import tpu as pltpu
from jax.experimental.pallas import tpu_sc as plsc

N_NODES_ = 10000
N_EDGES_ = 320000
D_ = 128
NW_ = 32
PER_TILE_ = N_EDGES_ // NW_
C_ = 48
NCHUNK_ = 210
NSLOT_ = 3
PAD_TILE_ = NCHUNK_ * C_
FULL_CHUNKS_ = PER_TILE_ // C_
TAIL_ = PER_TILE_ - FULL_CHUNKS_ * C_
EPS_ = 1e-8


def _norm_body(h_ref, n_ref):
    x = h_ref[...]
    n_ref[...] = jnp.sqrt(jnp.sum(x * x, axis=1))


def _node_norms(h):
    return pl.pallas_call(
        _norm_body,
        out_shape=jax.ShapeDtypeStruct((N_NODES_,), jnp.float32),
    )(h)


def _sc_body(h_hbm, src_hbm, dst_hbm, n_hbm, out_hbm,
             idx_s, idx_d, ntab, hsh, buf0, buf1, buf2, ob0, ob1, ob2,
             sb0, sb1, sb2, sa0, sa1, sa2, so0, so1, so2):
    cid = lax.axis_index("c")
    sid = lax.axis_index("s")
    wid = sid * 2 + cid
    base = wid * PER_TILE_

    rows_per_sub = 624
    rbase = sid * rows_per_sub
    pltpu.sync_copy(h_hbm.at[pl.ds(rbase, rows_per_sub)],
                    hsh.at[pl.ds(rbase, rows_per_sub)])

    @pl.when(sid == 0)
    def _():
        pltpu.sync_copy(h_hbm.at[pl.ds(9984, 16)], hsh.at[pl.ds(9984, 16)])

    pltpu.sync_copy(src_hbm.at[pl.ds(base, PER_TILE_)], idx_s.at[pl.ds(0, PER_TILE_)])
    pltpu.sync_copy(dst_hbm.at[pl.ds(base, PER_TILE_)], idx_d.at[pl.ds(0, PER_TILE_)])
    pltpu.sync_copy(n_hbm, ntab)

    zeros_i = jnp.zeros((16,), jnp.int32)
    for t in range((PAD_TILE_ - PER_TILE_) // 16):
        idx_s[pl.ds(PER_TILE_ + 16 * t, 16)] = zeros_i
        idx_d[pl.ds(PER_TILE_ + 16 * t, 16)] = zeros_i

    lane = lax.broadcasted_iota(jnp.int32, (16,), 0)
    bufs = (buf0, buf1, buf2)
    obufs = (ob0, ob1, ob2)
    semb = (sb0, sb1, sb2)
    sema = (sa0, sa1, sa2)
    semo = (so0, so1, so2)

    def fire_base(k, slot):
        pltpu.async_copy(hsh.at[idx_s.at[pl.ds(k * C_, C_)]],
                         bufs[slot], semb[slot])

    def fire_add(k, slot):
        pltpu.async_copy(hsh.at[idx_d.at[pl.ds(k * C_, C_)]],
                         bufs[slot], sema[slot], add=True)

    def wait_base(slot):
        pltpu.make_async_copy(h_hbm.at[pl.ds(0, C_)], bufs[slot],
                              semb[slot]).wait()

    def wait_add(slot):
        pltpu.make_async_copy(h_hbm.at[pl.ds(0, C_)], bufs[slot],
                              sema[slot]).wait()

    def fire_out(k, slot):
        pltpu.async_copy(obufs[slot], out_hbm.at[pl.ds(base + k * C_, C_)],
                         semo[slot])

    def wait_out(slot):
        pltpu.make_async_copy(obufs[slot], out_hbm.at[pl.ds(0, C_)],
                              semo[slot]).wait()

    def compute(k, slot):
        buf = bufs[slot]
        ob = obufs[slot]
        for g in range(C_ // 16):
            ssq16 = jnp.zeros((16,), jnp.float32)
            for e16 in range(16):
                ec = g * 16 + e16
                v = buf[ec, pl.ds(0, 16)]
                acc = v * v
                for j in range(1, 8):
                    v = buf[ec, pl.ds(16 * j, 16)]
                    acc = acc + v * v
                for sh in (8, 4, 2, 1):
                    acc = acc + acc.at[lane ^ sh].get(mode="promise_in_bounds")
                ssq16 = jnp.where(lane == e16, acc, ssq16)
            eoff = k * C_ + g * 16
            is16 = idx_s[pl.ds(eoff, 16)]
            id16 = idx_d[pl.ds(eoff, 16)]
            ns = plsc.load_gather(ntab, [is16])
            nd = plsc.load_gather(ntab, [id16])
            dot16 = (ssq16 - ns * ns - nd * nd) * 0.5
            ob[pl.ds(g * 16, 16)] = dot16 / jnp.maximum(ns * nd, EPS_)

    plsc.subcore_barrier()

    for p in range(NSLOT_):
        fire_base(p, p)
    wait_base(0)
    fire_add(0, 0)

    def ring_body(i, _):
        k0 = i * NSLOT_
        for b in range(NSLOT_):
            k = k0 + b
            nslot = (b + 1) % NSLOT_

            @pl.when(k + 1 < NCHUNK_)
            def _():
                wait_base(nslot)
                fire_add(k + 1, nslot)

            @pl.when(k >= NSLOT_)
            def _():
                wait_out(b)

            wait_add(b)
            compute(k, b)

            @pl.when(k < FULL_CHUNKS_)
            def _():
                fire_out(k, b)

            @pl.when(k + NSLOT_ < NCHUNK_)
            def _():
                fire_base(k + NSLOT_, b)
        return 0

    lax.fori_loop(0, NCHUNK_ // NSLOT_, ring_body, 0)

    pltpu.sync_copy(obufs[FULL_CHUNKS_ % NSLOT_].at[pl.ds(0, TAIL_)],
                    out_hbm.at[pl.ds(base + FULL_CHUNKS_ * C_, TAIL_)])

    wait_out((FULL_CHUNKS_ - 1) % NSLOT_)


def _edge_cos_sc(h, src, dst, n):
    mesh = plsc.VectorSubcoreMesh(core_axis_name="c", subcore_axis_name="s")
    f = functools.partial(
        pl.kernel,
        mesh=mesh,
        out_type=jax.ShapeDtypeStruct((N_EDGES_,), jnp.float32),
        scratch_types=[
            pltpu.VMEM((PAD_TILE_,), jnp.int32),
            pltpu.VMEM((PAD_TILE_,), jnp.int32),
            pltpu.VMEM((N_NODES_,), jnp.float32),
            pltpu.VMEM_SHARED((N_NODES_, D_), jnp.float32),
            pltpu.VMEM((C_, D_), jnp.float32),
            pltpu.VMEM((C_, D_), jnp.float32),
            pltpu.VMEM((C_, D_), jnp.float32),
            pltpu.VMEM((C_,), jnp.float32),
            pltpu.VMEM((C_,), jnp.float32),
            pltpu.VMEM((C_,), jnp.float32),
        ] + [pltpu.SemaphoreType.DMA] * 9,
        compiler_params=pltpu.CompilerParams(needs_layout_passes=False),
    )(_sc_body)
    return f(h, src, dst, n)


def kernel(h, edge_index):
    ei = edge_index.astype(jnp.int32)
    n = _node_norms(h)
    return _edge_cos_sc(h, ei[0], ei[1], n)
